# baseline (speedup 1.0000x reference)
import functools
import math

import jax
import jax.numpy as jnp
from jax.experimental import pallas as pl
from jax.experimental.pallas import tpu as pltpu

Q = 1024
D_IN = 512
D_EMB = 128
K_GAL = 20000

TAU = 1.75
ALPHA = 0.4
S_SCALE = math.sqrt(math.log2(math.e)) / (TAU * TAU)
S_EPS = S_SCALE * 1e-12
C_THIRD = -math.log2(3.0)
T_CUT = math.log2(ALPHA) + C_THIRD

KBLK = 2048


def _embed_kernel(x_ref, w1_ref, w2_ref, w3_ref, a1_ref, qc_ref):
    x = x_ref[...]
    e1 = jax.lax.dot_general(
        x, w1_ref[...], (((1,), (0,)), ((), ())),
        preferred_element_type=jnp.float32)
    q2 = jnp.sum(e1 * e1, axis=1, keepdims=True)
    ones = jnp.ones((Q, 1), jnp.float32)
    a1_ref[...] = jnp.concatenate(
        [e1 * (-2.0 * S_SCALE), S_SCALE * q2 + S_EPS, ones], axis=1)
    e2 = jax.lax.dot_general(
        x, w2_ref[...], (((1,), (0,)), ((), ())),
        preferred_element_type=jnp.float32)
    e3 = jax.lax.dot_general(
        x, w3_ref[...], (((1,), (0,)), ((), ())),
        preferred_element_type=jnp.float32)
    qn2 = e2 * ((1.0 / 3.0) / (jnp.sqrt(jnp.sum(e2 * e2, axis=1, keepdims=True)) + 1e-12))
    qn3 = e3 * ((1.0 / 3.0) / (jnp.sqrt(jnp.sum(e3 * e3, axis=1, keepdims=True)) + 1e-12))
    qc_ref[...] = jnp.concatenate([qn2, qn3], axis=1)


def _main_kernel(a1_ref, qc_ref, g1_ref, g2_ref, g3_ref, o_ref):
    g1 = g1_ref[...]
    g1b = S_SCALE * jnp.sum(g1 * g1, axis=1, keepdims=True)
    ones = jnp.ones((KBLK, 1), jnp.float32)
    ag1 = jnp.concatenate([g1, ones, g1b], axis=1)
    w = jax.lax.dot_general(
        a1_ref[...], ag1, (((1,), (1,)), ((), ())),
        preferred_element_type=jnp.float32)
    t = C_THIRD - w * w
    cer3 = jnp.where(t >= T_CUT, jnp.exp2(t), 0.0)

    g2 = g2_ref[...]
    g3 = g3_ref[...]
    r2 = 1.0 / (jnp.sqrt(jnp.sum(g2 * g2, axis=1, keepdims=True)) + 1e-12)
    r3 = 1.0 / (jnp.sqrt(jnp.sum(g3 * g3, axis=1, keepdims=True)) + 1e-12)
    gc = jnp.concatenate([g2 * r2, g3 * r3], axis=1)
    ccos = jax.lax.dot_general(
        qc_ref[...], gc, (((1,), (1,)), ((), ())),
        preferred_element_type=jnp.float32)
    o_ref[...] = cer3 + ccos


@functools.partial(jax.jit, static_argnames=("interpret",))
def kernel(x, W1, W2, W3, G1, G2, G3, interpret=False):
    a1, qc = pl.pallas_call(
        _embed_kernel,
        out_shape=[
            jax.ShapeDtypeStruct((Q, D_EMB + 2), jnp.float32),
            jax.ShapeDtypeStruct((Q, 2 * D_EMB), jnp.float32),
        ],
        interpret=interpret,
    )(x, W1, W2, W3)

    nblk = pl.cdiv(K_GAL, KBLK)
    gal_spec = pl.BlockSpec((KBLK, D_EMB), lambda k: (k, 0))
    out = pl.pallas_call(
        _main_kernel,
        grid=(nblk,),
        in_specs=[
            pl.BlockSpec((Q, D_EMB + 2), lambda k: (0, 0)),
            pl.BlockSpec((Q, 2 * D_EMB), lambda k: (0, 0)),
            gal_spec, gal_spec, gal_spec,
        ],
        out_specs=pl.BlockSpec((Q, KBLK), lambda k: (0, k)),
        out_shape=jax.ShapeDtypeStruct((Q, K_GAL), jnp.float32),
        compiler_params=pltpu.CompilerParams(
            dimension_semantics=("parallel",)),
        interpret=interpret,
    )(a1, qc, G1, G2, G3)
    return out


# speedup vs baseline: 1.0058x; 1.0058x over previous
import functools
import math

import jax
import jax.numpy as jnp
from jax.experimental import pallas as pl
from jax.experimental.pallas import tpu as pltpu

Q = 1024
D_IN = 512
D_EMB = 128
K_GAL = 20000

TAU = 1.75
ALPHA = 0.4
S_SCALE = math.sqrt(math.log2(math.e)) / (TAU * TAU)
S_EPS = S_SCALE * 1e-12
C_THIRD = -math.log2(3.0)
T_CUT = math.log2(ALPHA) + C_THIRD

KBLK = 1024


def _main_kernel(x_ref, w1_ref, w2_ref, w3_ref, g1_ref, g2_ref, g3_ref,
                 o_ref, e1s_s, qb_s, qc_s):
    @pl.when(pl.program_id(0) == 0)
    def _embed():
        x = x_ref[...]
        e1 = jax.lax.dot_general(
            x, w1_ref[...], (((1,), (0,)), ((), ())),
            preferred_element_type=jnp.float32)
        q2 = jnp.sum(e1 * e1, axis=1, keepdims=True)
        e1s_s[...] = e1 * (-2.0 * S_SCALE)
        qb_s[...] = S_SCALE * q2 + S_EPS
        e2 = jax.lax.dot_general(
            x, w2_ref[...], (((1,), (0,)), ((), ())),
            preferred_element_type=jnp.float32)
        e3 = jax.lax.dot_general(
            x, w3_ref[...], (((1,), (0,)), ((), ())),
            preferred_element_type=jnp.float32)
        qn2 = e2 * ((1.0 / 3.0) / (jnp.sqrt(jnp.sum(e2 * e2, axis=1, keepdims=True)) + 1e-12))
        qn3 = e3 * ((1.0 / 3.0) / (jnp.sqrt(jnp.sum(e3 * e3, axis=1, keepdims=True)) + 1e-12))
        qc_s[...] = jnp.concatenate([qn2, qn3], axis=1)

    g1 = g1_ref[...]
    g1b = S_SCALE * jnp.sum(g1 * g1, axis=1)[None, :]
    m0 = jax.lax.dot_general(
        e1s_s[...], g1, (((1,), (1,)), ((), ())),
        preferred_element_type=jnp.float32)
    w = m0 + qb_s[...] + g1b
    t = C_THIRD - w * w
    cer3 = jnp.where(t >= T_CUT, jnp.exp2(t), 0.0)

    g2 = g2_ref[...]
    g3 = g3_ref[...]
    r2 = 1.0 / (jnp.sqrt(jnp.sum(g2 * g2, axis=1, keepdims=True)) + 1e-12)
    r3 = 1.0 / (jnp.sqrt(jnp.sum(g3 * g3, axis=1, keepdims=True)) + 1e-12)
    gc = jnp.concatenate([g2 * r2, g3 * r3], axis=1)
    ccos = jax.lax.dot_general(
        qc_s[...], gc, (((1,), (1,)), ((), ())),
        preferred_element_type=jnp.float32)
    o_ref[...] = cer3 + ccos


@functools.partial(jax.jit, static_argnames=("interpret",))
def kernel(x, W1, W2, W3, G1, G2, G3, interpret=False):
    nblk = pl.cdiv(K_GAL, KBLK)
    gal_spec = pl.BlockSpec((KBLK, D_EMB), lambda k: (k, 0))
    const2d = lambda shape: pl.BlockSpec(shape, lambda k: (0, 0))
    out = pl.pallas_call(
        _main_kernel,
        grid=(nblk,),
        in_specs=[
            const2d((Q, D_IN)),
            const2d((D_IN, D_EMB)),
            const2d((D_IN, D_EMB)),
            const2d((D_IN, D_EMB)),
            gal_spec, gal_spec, gal_spec,
        ],
        out_specs=pl.BlockSpec((Q, KBLK), lambda k: (0, k)),
        out_shape=jax.ShapeDtypeStruct((Q, K_GAL), jnp.float32),
        scratch_shapes=[
            pltpu.VMEM((Q, D_EMB), jnp.float32),
            pltpu.VMEM((Q, 1), jnp.float32),
            pltpu.VMEM((Q, 2 * D_EMB), jnp.float32),
        ],
        interpret=interpret,
    )(x, W1, W2, W3, G1, G2, G3)
    return out


# speedup vs baseline: 1.0373x; 1.0313x over previous
import functools
import math

import jax
import jax.numpy as jnp
from jax.experimental import pallas as pl
from jax.experimental.pallas import tpu as pltpu

Q = 1024
D_IN = 512
D_EMB = 128
K_GAL = 20000

TAU = 1.75
ALPHA = 0.4
S_SCALE = math.sqrt(math.log2(math.e)) / (TAU * TAU)
S_EPS = S_SCALE * 1e-12
C_THIRD = -math.log2(3.0)
T_CUT = math.log2(ALPHA) + C_THIRD

KBLK = 2048


def _main_kernel(x_ref, w1_ref, w2_ref, w3_ref, g1_ref, g2_ref, g3_ref,
                 o_ref, e1s_s, qb_s, qc_s):
    @pl.when(pl.program_id(0) == 0)
    def _embed():
        x = x_ref[...]
        e1 = jax.lax.dot_general(
            x, w1_ref[...], (((1,), (0,)), ((), ())),
            preferred_element_type=jnp.float32)
        q2 = jnp.sum(e1 * e1, axis=1, keepdims=True)
        e1s_s[...] = e1 * (-2.0 * S_SCALE)
        qb_s[...] = S_SCALE * q2 + S_EPS
        e2 = jax.lax.dot_general(
            x, w2_ref[...], (((1,), (0,)), ((), ())),
            preferred_element_type=jnp.float32)
        e3 = jax.lax.dot_general(
            x, w3_ref[...], (((1,), (0,)), ((), ())),
            preferred_element_type=jnp.float32)
        qn2 = e2 * ((1.0 / 3.0) / (jnp.sqrt(jnp.sum(e2 * e2, axis=1, keepdims=True)) + 1e-12))
        qn3 = e3 * ((1.0 / 3.0) / (jnp.sqrt(jnp.sum(e3 * e3, axis=1, keepdims=True)) + 1e-12))
        qc_s[...] = jnp.concatenate([qn2, qn3], axis=1)

    g1 = g1_ref[...]
    g1b = S_SCALE * jnp.sum(g1 * g1, axis=1)[None, :]
    m0 = jax.lax.dot_general(
        e1s_s[...], g1, (((1,), (1,)), ((), ())),
        preferred_element_type=jnp.float32)
    w = m0 + qb_s[...] + g1b
    t = C_THIRD - w * w
    cer3 = jnp.where(t >= T_CUT, jnp.exp2(t), 0.0)

    g2 = g2_ref[...]
    g3 = g3_ref[...]
    r2 = 1.0 / (jnp.sqrt(jnp.sum(g2 * g2, axis=1, keepdims=True)) + 1e-12)
    r3 = 1.0 / (jnp.sqrt(jnp.sum(g3 * g3, axis=1, keepdims=True)) + 1e-12)
    gc = jnp.concatenate([g2 * r2, g3 * r3], axis=1)
    ccos = jax.lax.dot_general(
        qc_s[...], gc, (((1,), (1,)), ((), ())),
        preferred_element_type=jnp.float32)
    o_ref[...] = cer3 + ccos


@functools.partial(jax.jit, static_argnames=("interpret",))
def kernel(x, W1, W2, W3, G1, G2, G3, interpret=False):
    nblk = pl.cdiv(K_GAL, KBLK)
    gal_spec = pl.BlockSpec((KBLK, D_EMB), lambda k: (k, 0))
    const2d = lambda shape: pl.BlockSpec(shape, lambda k: (0, 0))
    out = pl.pallas_call(
        _main_kernel,
        grid=(nblk,),
        in_specs=[
            const2d((Q, D_IN)),
            const2d((D_IN, D_EMB)),
            const2d((D_IN, D_EMB)),
            const2d((D_IN, D_EMB)),
            gal_spec, gal_spec, gal_spec,
        ],
        out_specs=pl.BlockSpec((Q, KBLK), lambda k: (0, k)),
        out_shape=jax.ShapeDtypeStruct((Q, K_GAL), jnp.float32),
        scratch_shapes=[
            pltpu.VMEM((Q, D_EMB), jnp.float32),
            pltpu.VMEM((Q, 1), jnp.float32),
            pltpu.VMEM((Q, 2 * D_EMB), jnp.float32),
        ],
        interpret=interpret,
    )(x, W1, W2, W3, G1, G2, G3)
    return out


# speedup vs baseline: 1.0391x; 1.0017x over previous
import functools
import math

import jax
import jax.numpy as jnp
from jax.experimental import pallas as pl
from jax.experimental.pallas import tpu as pltpu

Q = 1024
D_IN = 512
D_EMB = 128
K_GAL = 20000

TAU = 1.75
ALPHA = 0.4
S_SCALE = math.sqrt(math.log2(math.e)) / (TAU * TAU)
S_EPS = S_SCALE * 1e-12
C_THIRD = -math.log2(3.0)
T_CUT = math.log2(ALPHA) + C_THIRD

KBLK = 2560


def _main_kernel(x_ref, w1_ref, w2_ref, w3_ref, g1_ref, g2_ref, g3_ref,
                 o_ref, e1s_s, qb_s, qc_s):
    @pl.when(pl.program_id(0) == 0)
    def _embed():
        x = x_ref[...]
        e1 = jax.lax.dot_general(
            x, w1_ref[...], (((1,), (0,)), ((), ())),
            preferred_element_type=jnp.float32)
        q2 = jnp.sum(e1 * e1, axis=1, keepdims=True)
        e1s_s[...] = e1 * (-2.0 * S_SCALE)
        qb_s[...] = S_SCALE * q2 + S_EPS
        e2 = jax.lax.dot_general(
            x, w2_ref[...], (((1,), (0,)), ((), ())),
            preferred_element_type=jnp.float32)
        e3 = jax.lax.dot_general(
            x, w3_ref[...], (((1,), (0,)), ((), ())),
            preferred_element_type=jnp.float32)
        qn2 = e2 * ((1.0 / 3.0) / (jnp.sqrt(jnp.sum(e2 * e2, axis=1, keepdims=True)) + 1e-12))
        qn3 = e3 * ((1.0 / 3.0) / (jnp.sqrt(jnp.sum(e3 * e3, axis=1, keepdims=True)) + 1e-12))
        qc_s[...] = jnp.concatenate([qn2, qn3], axis=1)

    g1 = g1_ref[...]
    g1b = S_SCALE * jnp.sum(g1 * g1, axis=1)[None, :]
    m0 = jax.lax.dot_general(
        e1s_s[...], g1, (((1,), (1,)), ((), ())),
        preferred_element_type=jnp.float32)
    w = m0 + qb_s[...] + g1b
    t = C_THIRD - w * w
    cer3 = jnp.where(t >= T_CUT, jnp.exp2(t), 0.0)

    g2 = g2_ref[...]
    g3 = g3_ref[...]
    r2 = 1.0 / (jnp.sqrt(jnp.sum(g2 * g2, axis=1, keepdims=True)) + 1e-12)
    r3 = 1.0 / (jnp.sqrt(jnp.sum(g3 * g3, axis=1, keepdims=True)) + 1e-12)
    gc = jnp.concatenate([g2 * r2, g3 * r3], axis=1)
    ccos = jax.lax.dot_general(
        qc_s[...], gc, (((1,), (1,)), ((), ())),
        preferred_element_type=jnp.float32)
    o_ref[...] = cer3 + ccos


@functools.partial(jax.jit, static_argnames=("interpret",))
def kernel(x, W1, W2, W3, G1, G2, G3, interpret=False):
    nblk = pl.cdiv(K_GAL, KBLK)
    gal_spec = pl.BlockSpec((KBLK, D_EMB), lambda k: (k, 0))
    const2d = lambda shape: pl.BlockSpec(shape, lambda k: (0, 0))
    out = pl.pallas_call(
        _main_kernel,
        grid=(nblk,),
        in_specs=[
            const2d((Q, D_IN)),
            const2d((D_IN, D_EMB)),
            const2d((D_IN, D_EMB)),
            const2d((D_IN, D_EMB)),
            gal_spec, gal_spec, gal_spec,
        ],
        out_specs=pl.BlockSpec((Q, KBLK), lambda k: (0, k)),
        out_shape=jax.ShapeDtypeStruct((Q, K_GAL), jnp.float32),
        scratch_shapes=[
            pltpu.VMEM((Q, D_EMB), jnp.float32),
            pltpu.VMEM((Q, 1), jnp.float32),
            pltpu.VMEM((Q, 2 * D_EMB), jnp.float32),
        ],
        interpret=interpret,
    )(x, W1, W2, W3, G1, G2, G3)
    return out
